# baseline (speedup 1.0000x reference)
import functools

import jax
import jax.numpy as jnp
from jax import lax
from jax.experimental import pallas as pl
from jax.experimental.pallas import tpu as pltpu
from jax.experimental.pallas import tpu_sc as plsc

B = 1024
F = 26
C = 1000
D = F * C
CH = 40
CPF = C // CH
MAGIC = -(-(1 << 20) // CPF)
NCHUNK = D // CH
NC = 2
NS = 16
NW = NC * NS
MAXC = -(-NCHUNK // NW)
L = 16


def _onehot_body(in_hbm, out_hbm, idx_v, buf0, buf1, sem0, sem1):
    wid = lax.axis_index("s") * NC + lax.axis_index("c")

    idx_copy = pltpu.async_copy(in_hbm, idx_v, sem0)

    zvec = jnp.zeros((L,), jnp.float32)
    UNR = 8

    def zero_buf(buf):
        def zrow(r, carry):
            def zcol(c, carry2):
                for k in range(UNR):
                    off = pl.multiple_of((c * UNR + k) * L, L)
                    buf[r, pl.ds(off, L)] = zvec
                return carry2
            lax.fori_loop(0, B // (L * UNR), zcol, 0)
            return carry
        lax.fori_loop(0, CH, zrow, 0)

    iota = lax.iota(jnp.int32, L)
    ones = jnp.ones((L,), jnp.float32)

    s = (NCHUNK * wid) >> 5
    e = (NCHUNK * (wid + 1)) >> 5

    def sweep(m, buf, val):
        j = (m * MAGIC) >> 20
        c0 = (m - CPF * j) * CH

        def body(i, carry):
            for k in range(4):
                off = pl.multiple_of((i * 4 + k) * L, L)
                v = idx_v[j, pl.ds(off, L)]
                crel = v - c0
                mask = (crel >= 0) & (crel < CH)
                crel_c = jnp.minimum(jnp.maximum(crel, 0), CH - 1)
                rvec = iota + (i * 4 + k) * L
                plsc.store_scatter(buf, [crel_c, rvec], val, mask=mask)
            return carry

        lax.fori_loop(0, B // (L * 4), body, 0)

    def row0_of(m):
        j = (m * MAGIC) >> 20
        c0 = (m - CPF * j) * CH
        return j * C + c0

    bufs = (buf0, buf1)
    sems = (sem0, sem1)
    nmin = NCHUNK // NW

    SPLIT = 24

    def start_dma(m, buf, sem):
        r0 = row0_of(m)
        pltpu.async_copy(
            buf.at[pl.ds(0, SPLIT)], out_hbm.at[pl.ds(r0, SPLIT)], sem)
        pltpu.async_copy(
            buf.at[pl.ds(SPLIT, CH - SPLIT)],
            out_hbm.at[pl.ds(r0 + SPLIT, CH - SPLIT)], sem)

    def drain(buf, sem):
        pltpu.make_async_copy(
            buf.at[pl.ds(0, SPLIT)], out_hbm.at[pl.ds(0, SPLIT)], sem).wait()
        pltpu.make_async_copy(
            buf.at[pl.ds(SPLIT, CH - SPLIT)],
            out_hbm.at[pl.ds(0, CH - SPLIT)], sem).wait()

    zero_buf(buf0)
    idx_copy.wait()
    sweep(s, buf0, ones)
    start_dma(s, buf0, sem0)
    zero_buf(buf1)
    sweep(s + 1, buf1, ones)
    start_dma(s + 1, buf1, sem1)

    def pair(i, carry):
        for b in (0, 1):
            m = s + 2 * i + b
            buf, sem = bufs[b], sems[b]
            drain(buf, sem)
            sweep(m - 2, buf, zvec)
            sweep(m, buf, ones)
            start_dma(m, buf, sem)
        return carry

    lax.fori_loop(1, nmin // 2, pair, 0)

    @pl.when(s + nmin < e)
    def _():
        drain(buf0, sem0)
        sweep(s + nmin - 2, buf0, zvec)
        sweep(s + nmin, buf0, ones)
        start_dma(s + nmin, buf0, sem0)

    drain(buf0, sem0)
    drain(buf1, sem1)


@jax.jit
def kernel(inputs):
    idx_t = inputs.astype(jnp.int32).T
    mesh = plsc.VectorSubcoreMesh(core_axis_name="c", subcore_axis_name="s")
    run = pl.kernel(
        _onehot_body,
        out_type=jax.ShapeDtypeStruct((D, B), jnp.float32),
        mesh=mesh,
        compiler_params=pltpu.CompilerParams(
            needs_layout_passes=False, use_tc_tiling_on_sc=True),
        scratch_types=[
            pltpu.VMEM((F, B), jnp.int32),
            pltpu.VMEM((CH, B), jnp.float32),
            pltpu.VMEM((CH, B), jnp.float32),
            pltpu.SemaphoreType.DMA,
            pltpu.SemaphoreType.DMA,
        ],
    )
    return run(idx_t).T


# speedup vs baseline: 1.0006x; 1.0006x over previous
import functools

import jax
import jax.numpy as jnp
from jax import lax
from jax.experimental import pallas as pl
from jax.experimental.pallas import tpu as pltpu
from jax.experimental.pallas import tpu_sc as plsc

B = 1024
F = 26
C = 1000
D = F * C
CH = 40
CPF = C // CH
MAGIC = -(-(1 << 20) // CPF)
NCHUNK = D // CH
NC = 2
NS = 16
NW = NC * NS
MAXC = -(-NCHUNK // NW)
L = 16


def _onehot_body(in_hbm, out_hbm, idx_v, buf0, buf1, sem0, sem1):
    wid = lax.axis_index("s") * NC + lax.axis_index("c")

    idx_copy = pltpu.async_copy(in_hbm, idx_v, sem0)

    zvec = jnp.zeros((L,), jnp.float32)
    UNR = 4

    def zero_buf(buf):
        def zrow(r, carry):
            def zcol(c, carry2):
                for k in range(UNR):
                    off = pl.multiple_of((c * UNR + k) * L, L)
                    buf[r, pl.ds(off, L)] = zvec
                return carry2
            lax.fori_loop(0, B // (L * UNR), zcol, 0)
            return carry
        lax.fori_loop(0, CH, zrow, 0)

    iota = lax.iota(jnp.int32, L)
    ones = jnp.ones((L,), jnp.float32)

    s = (NCHUNK * wid) >> 5
    e = (NCHUNK * (wid + 1)) >> 5

    def sweep(m, buf, val):
        j = (m * MAGIC) >> 20
        c0 = (m - CPF * j) * CH

        def body(i, carry):
            off = pl.multiple_of(i * L, L)
            v = idx_v[j, pl.ds(off, L)]
            crel = v - c0
            mask = (crel >= 0) & (crel < CH)
            crel_c = jnp.minimum(jnp.maximum(crel, 0), CH - 1)
            rvec = iota + i * L
            plsc.store_scatter(buf, [crel_c, rvec], val, mask=mask)
            return carry

        lax.fori_loop(0, B // L, body, 0)

    def row0_of(m):
        j = (m * MAGIC) >> 20
        c0 = (m - CPF * j) * CH
        return j * C + c0

    bufs = (buf0, buf1)
    sems = (sem0, sem1)
    nmin = NCHUNK // NW

    def start_dma(m, buf, sem):
        pltpu.async_copy(buf, out_hbm.at[pl.ds(row0_of(m), CH)], sem)

    def drain(buf, sem):
        pltpu.make_async_copy(buf, out_hbm.at[pl.ds(0, CH)], sem).wait()

    zero_buf(buf0)
    idx_copy.wait()
    sweep(s, buf0, ones)
    start_dma(s, buf0, sem0)
    zero_buf(buf1)
    sweep(s + 1, buf1, ones)
    start_dma(s + 1, buf1, sem1)

    def pair(i, carry):
        for b in (0, 1):
            m = s + 2 * i + b
            buf, sem = bufs[b], sems[b]
            drain(buf, sem)
            sweep(m - 2, buf, zvec)
            sweep(m, buf, ones)
            start_dma(m, buf, sem)
        return carry

    lax.fori_loop(1, nmin // 2, pair, 0)

    @pl.when(s + nmin < e)
    def _():
        drain(buf0, sem0)
        sweep(s + nmin - 2, buf0, zvec)
        sweep(s + nmin, buf0, ones)
        start_dma(s + nmin, buf0, sem0)

    drain(buf0, sem0)
    drain(buf1, sem1)


@jax.jit
def kernel(inputs):
    idx_t = inputs.astype(jnp.int32).T
    mesh = plsc.VectorSubcoreMesh(core_axis_name="c", subcore_axis_name="s")
    run = pl.kernel(
        _onehot_body,
        out_type=jax.ShapeDtypeStruct((D, B), jnp.float32),
        mesh=mesh,
        compiler_params=pltpu.CompilerParams(
            needs_layout_passes=False, use_tc_tiling_on_sc=True),
        scratch_types=[
            pltpu.VMEM((F, B), jnp.int32),
            pltpu.VMEM((CH, B), jnp.float32),
            pltpu.VMEM((CH, B), jnp.float32),
            pltpu.SemaphoreType.DMA,
            pltpu.SemaphoreType.DMA,
        ],
    )
    return run(idx_t).T


# speedup vs baseline: 1.0092x; 1.0086x over previous
import jax
import jax.numpy as jnp
from jax import lax
from jax.experimental import pallas as pl
from jax.experimental.pallas import tpu as pltpu
from jax.experimental.pallas import tpu_sc as plsc

B = 1024
F = 26
C = 1000
D = F * C
CH = 40
CPF = C // CH
MAGIC = -(-(1 << 20) // CPF)
NCHUNK = D // CH
NC = 2
NS = 16
NW = NC * NS
MAXC = -(-NCHUNK // NW)
L = 16


def _onehot_body(in_hbm, out_hbm, idx_v, buf0, buf1, sem0, sem1):
    wid = lax.axis_index("s") * NC + lax.axis_index("c")

    idx_copy = pltpu.async_copy(in_hbm, idx_v, sem0)

    zvec = jnp.zeros((L,), jnp.float32)
    UNR = 8

    def zero_buf(buf):
        def zrow(r, carry):
            def zcol(c, carry2):
                for k in range(UNR):
                    off = pl.multiple_of((c * UNR + k) * L, L)
                    buf[r, pl.ds(off, L)] = zvec
                return carry2
            lax.fori_loop(0, B // (L * UNR), zcol, 0)
            return carry
        lax.fori_loop(0, CH, zrow, 0)

    iota = lax.iota(jnp.int32, L)
    ones = jnp.ones((L,), jnp.float32)

    s = (NCHUNK * wid) >> 5
    e = (NCHUNK * (wid + 1)) >> 5

    def sweep(m, buf, val):
        j = (m * MAGIC) >> 20
        c0 = (m - CPF * j) * CH

        def body(i, carry):
            off = pl.multiple_of(i * L, L)
            v = idx_v[j, pl.ds(off, L)]
            crel = v - c0
            mask = (crel >= 0) & (crel < CH)
            crel_c = jnp.minimum(jnp.maximum(crel, 0), CH - 1)
            rvec = iota + i * L
            plsc.store_scatter(buf, [crel_c, rvec], val, mask=mask)
            return carry

        lax.fori_loop(0, B // L, body, 0)

    def row0_of(m):
        j = (m * MAGIC) >> 20
        c0 = (m - CPF * j) * CH
        return j * C + c0

    bufs = (buf0, buf1)
    sems = (sem0, sem1)
    nmin = NCHUNK // NW

    def start_dma(m, buf, sem):
        pltpu.async_copy(buf, out_hbm.at[pl.ds(row0_of(m), CH)], sem)

    def drain(buf, sem):
        pltpu.make_async_copy(buf, out_hbm.at[pl.ds(0, CH)], sem).wait()

    zero_buf(buf0)
    idx_copy.wait()
    sweep(s, buf0, ones)
    start_dma(s, buf0, sem0)
    zero_buf(buf1)
    sweep(s + 1, buf1, ones)
    start_dma(s + 1, buf1, sem1)

    def pair(i, carry):
        for b in (0, 1):
            m = s + 2 * i + b
            buf, sem = bufs[b], sems[b]
            drain(buf, sem)
            sweep(m - 2, buf, zvec)
            sweep(m, buf, ones)
            start_dma(m, buf, sem)
        return carry

    lax.fori_loop(1, nmin // 2, pair, 0)

    @pl.when(s + nmin < e)
    def _():
        drain(buf0, sem0)
        sweep(s + nmin - 2, buf0, zvec)
        sweep(s + nmin, buf0, ones)
        start_dma(s + nmin, buf0, sem0)

    drain(buf0, sem0)
    drain(buf1, sem1)


@jax.jit
def kernel(inputs):
    idx_t = inputs.astype(jnp.int32).T
    mesh = plsc.VectorSubcoreMesh(core_axis_name="c", subcore_axis_name="s")
    run = pl.kernel(
        _onehot_body,
        out_type=jax.ShapeDtypeStruct((D, B), jnp.float32),
        mesh=mesh,
        compiler_params=pltpu.CompilerParams(
            needs_layout_passes=False, use_tc_tiling_on_sc=True),
        scratch_types=[
            pltpu.VMEM((F, B), jnp.int32),
            pltpu.VMEM((CH, B), jnp.float32),
            pltpu.VMEM((CH, B), jnp.float32),
            pltpu.SemaphoreType.DMA,
            pltpu.SemaphoreType.DMA,
        ],
    )
    return run(idx_t).T


# speedup vs baseline: 1.0121x; 1.0029x over previous
import jax
import jax.numpy as jnp
from jax import lax
from jax.experimental import pallas as pl
from jax.experimental.pallas import tpu as pltpu
from jax.experimental.pallas import tpu_sc as plsc

B = 1024
F = 26
C = 1000
D = F * C
CH = 40
CPF = C // CH
MAGIC = -(-(1 << 20) // CPF)
NCHUNK = D // CH
NC = 2
NS = 16
NW = NC * NS
L = 16


def _onehot_body(in_hbm, out_hbm, idx_v, buf0, buf1, sem0, sem1):
    wid = lax.axis_index("s") * NC + lax.axis_index("c")

    idx_copy = pltpu.async_copy(in_hbm, idx_v, sem0)

    zvec = jnp.zeros((L,), jnp.float32)
    UNR = 8

    def zero_buf(buf):
        def zrow(r, carry):
            def zcol(c, carry2):
                for k in range(UNR):
                    off = pl.multiple_of((c * UNR + k) * L, L)
                    buf[r, pl.ds(off, L)] = zvec
                return carry2
            lax.fori_loop(0, B // (L * UNR), zcol, 0)
            return carry
        lax.fori_loop(0, CH, zrow, 0)

    iota = lax.iota(jnp.int32, L)
    ones = jnp.ones((L,), jnp.float32)

    s = (NCHUNK * wid) >> 5
    e = (NCHUNK * (wid + 1)) >> 5

    def sweep(m, buf, val):
        j = (m * MAGIC) >> 20
        c0 = (m - CPF * j) * CH

        def body(i, carry):
            off = pl.multiple_of(i * L, L)
            v = idx_v[j, pl.ds(off, L)]
            crel = v - c0
            mask = (crel >= 0) & (crel < CH)
            crel_c = jnp.minimum(jnp.maximum(crel, 0), CH - 1)
            rvec = iota + i * L
            plsc.store_scatter(buf, [crel_c, rvec], val, mask=mask)
            return carry

        lax.fori_loop(0, B // L, body, 0)

    def row0_of(m):
        j = (m * MAGIC) >> 20
        c0 = (m - CPF * j) * CH
        return j * C + c0

    bufs = (buf0, buf1)
    sems = (sem0, sem1)
    nmin = NCHUNK // NW

    def start_dma(m, buf, sem):
        pltpu.async_copy(buf, out_hbm.at[pl.ds(row0_of(m), CH)], sem)

    def drain(buf, sem):
        pltpu.make_async_copy(buf, out_hbm.at[pl.ds(0, CH)], sem).wait()

    zero_buf(buf0)
    idx_copy.wait()
    sweep(s, buf0, ones)
    start_dma(s, buf0, sem0)
    zero_buf(buf1)
    sweep(s + 1, buf1, ones)
    start_dma(s + 1, buf1, sem1)

    def pair(i, carry):
        for b in (0, 1):
            m = s + 2 * i + b
            buf, sem = bufs[b], sems[b]
            drain(buf, sem)
            sweep(m - 2, buf, zvec)
            sweep(m, buf, ones)
            start_dma(m, buf, sem)
        return carry

    lax.fori_loop(1, nmin // 2, pair, 0)

    @pl.when(s + nmin < e)
    def _():
        drain(buf0, sem0)
        sweep(s + nmin - 2, buf0, zvec)
        sweep(s + nmin, buf0, ones)
        start_dma(s + nmin, buf0, sem0)

    drain(buf0, sem0)
    drain(buf1, sem1)


@jax.jit
def kernel(inputs):
    idx_t = inputs.astype(jnp.int32).T
    mesh = plsc.VectorSubcoreMesh(core_axis_name="c", subcore_axis_name="s")
    run = pl.kernel(
        _onehot_body,
        out_type=jax.ShapeDtypeStruct((D, B), jnp.float32),
        mesh=mesh,
        compiler_params=pltpu.CompilerParams(
            needs_layout_passes=False, use_tc_tiling_on_sc=True),
        scratch_types=[
            pltpu.VMEM((F, B), jnp.int32),
            pltpu.VMEM((CH, B), jnp.float32),
            pltpu.VMEM((CH, B), jnp.float32),
            pltpu.SemaphoreType.DMA,
            pltpu.SemaphoreType.DMA,
        ],
    )
    return run(idx_t).T
